# baseline (speedup 1.0000x reference)
import functools

import jax
import jax.numpy as jnp
from jax import lax
from jax.experimental import pallas as pl
from jax.experimental.pallas import tpu as pltpu
from jax.experimental.pallas import tpu_sc as plsc

N_USER = 27094
M_SPOT = 42852
HIDDEN = 64
HH = 32
NUM_LAYERS = 3
N_EDGES = 1000000

N_PAD = 27136
M_PAD = 43008
N_PER_TILE = N_PAD // 16
M_PER_TILE = M_PAD // 16

ROWS_PER_CHUNK = 8
CHUNK = ROWS_PER_CHUNK * 128
NCHUNK = -(-N_EDGES // CHUNK)
E_PAD = NCHUNK * CHUNK
NIDXROWS = NCHUNK * ROWS_PER_CHUNK
ITERS_PER_TILE = -(-NCHUNK // 16)
ZROWS = CHUNK

_mesh = plsc.VectorSubcoreMesh(core_axis_name="c", subcore_axis_name="s")
_sc_params = pltpu.CompilerParams(use_tc_tiling_on_sc=False)
_sc_params_nl = pltpu.CompilerParams(use_tc_tiling_on_sc=False,
                                     needs_layout_passes=False)


def _counts_body(ru_hbm, rs_hbm, du_out, ds_out, idx_v, ones_v, zb_v, acc, sem):
    c = lax.axis_index("c")
    s = lax.axis_index("s")

    def fill(ref, n, val):
        def b(i, _):
            ref[pl.ds(i * 16, 16)] = jnp.full((16,), val, jnp.float32)
            return 0
        lax.fori_loop(0, n // 16, b, 0)

    fill(ones_v, 128, 1.0)
    fill(zb_v, M_PER_TILE, 0.0)

    def count_into(idx_hbm, out_hbm, per_tile):
        pltpu.sync_copy(zb_v.at[pl.ds(0, per_tile)],
                        acc.at[pl.ds(s * per_tile, per_tile)])
        plsc.subcore_barrier()

        def chunk_body(j, _):
            k = s + 16 * j

            @pl.when(k < NCHUNK)
            def _():
                pltpu.sync_copy(
                    idx_hbm.at[pl.ds(k * ROWS_PER_CHUNK, ROWS_PER_CHUNK)],
                    idx_v)
                for r in range(ROWS_PER_CHUNK):
                    pltpu.sync_copy(ones_v, acc.at[idx_v.at[r]], add=True)
            return 0

        lax.fori_loop(0, ITERS_PER_TILE, chunk_body, 0)
        plsc.subcore_barrier()
        off = 0
        while off < per_tile:
            nd = min(M_PER_TILE, per_tile - off)
            pltpu.sync_copy(acc.at[pl.ds(s * per_tile + off, nd)],
                            zb_v.at[pl.ds(0, nd)])
            pltpu.sync_copy(zb_v.at[pl.ds(0, nd)],
                            out_hbm.at[pl.ds(s * per_tile + off, nd)])
            off += nd

    @pl.when(c == 0)
    def _():
        count_into(ru_hbm, du_out, N_PER_TILE)

    @pl.when(c == 1)
    def _():
        count_into(rs_hbm, ds_out, M_PER_TILE)


_counts_call = functools.partial(
    pl.kernel,
    out_type=(jax.ShapeDtypeStruct((N_PAD,), jnp.float32),
              jax.ShapeDtypeStruct((M_PAD,), jnp.float32)),
    mesh=_mesh,
    scratch_types=[
        pltpu.VMEM((ROWS_PER_CHUNK, 128), jnp.int32),
        pltpu.VMEM((128,), jnp.float32),
        pltpu.VMEM((M_PER_TILE,), jnp.float32),
        pltpu.VMEM_SHARED((M_PAD,), jnp.float32),
        pltpu.SemaphoreType.DMA,
    ],
    compiler_params=_sc_params,
)(_counts_body)


def _layer_body(ru_hbm, rs_hbm, ss_lo, ss_hi, su_lo, su_hi,
                inv_u_hbm, inv_s_hbm, zeros2d,
                rawu_lo, rawu_hi, raws_lo, raws_hi,
                sun_lo, sun_hi, ssn_lo, ssn_hi,
                isrc_a, idst_a, isrc_b, idst_b, rows_v, inv_v,
                acc, semg, sems, semi):
    c = lax.axis_index("c")
    s = lax.axis_index("s")
    GR = ROWS_PER_CHUNK // 2

    def direction(src_tbl, isrc_hbm, idst_hbm, out_hbm, inv_hbm, scaled_hbm,
                  per_tile):
        pltpu.sync_copy(zeros2d, rows_v)
        off = 0
        while off < per_tile:
            nz = min(CHUNK, per_tile - off)
            pltpu.sync_copy(rows_v.at[pl.ds(0, nz)],
                            acc.at[pl.ds(s * per_tile + off, nz)])
            off += nz
        plsc.subcore_barrier()

        def load_idx(k, isrc_v, idst_v):
            pltpu.sync_copy(
                isrc_hbm.at[pl.ds(k * ROWS_PER_CHUNK, ROWS_PER_CHUNK)],
                isrc_v)
            pltpu.sync_copy(
                idst_hbm.at[pl.ds(k * ROWS_PER_CHUNK, ROWS_PER_CHUNK)],
                idst_v)

        def load_idx_async(k, isrc_v, idst_v):
            pltpu.async_copy(
                isrc_hbm.at[pl.ds(k * ROWS_PER_CHUNK, ROWS_PER_CHUNK)],
                isrc_v, semi)
            pltpu.async_copy(
                idst_hbm.at[pl.ds(k * ROWS_PER_CHUNK, ROWS_PER_CHUNK)],
                idst_v, semi)

        def load_idx_wait(k, isrc_v, idst_v):
            pltpu.make_async_copy(
                isrc_hbm.at[pl.ds(k * ROWS_PER_CHUNK, ROWS_PER_CHUNK)],
                isrc_v, semi).wait()
            pltpu.make_async_copy(
                idst_hbm.at[pl.ds(k * ROWS_PER_CHUNK, ROWS_PER_CHUNK)],
                idst_v, semi).wait()

        def gath(isrc_v, r0):
            for r in range(r0, r0 + GR):
                pltpu.async_copy(src_tbl.at[isrc_v.at[r]],
                                 rows_v.at[pl.ds(r * 128, 128)], semg)

        def gath_wait(isrc_v, r0):
            for r in range(r0, r0 + GR):
                pltpu.make_async_copy(src_tbl.at[isrc_v.at[r]],
                                      rows_v.at[pl.ds(r * 128, 128)],
                                      semg).wait()

        def scat(idst_v, r0):
            for r in range(r0, r0 + GR):
                pltpu.async_copy(rows_v.at[pl.ds(r * 128, 128)],
                                 acc.at[idst_v.at[r]], sems, add=True)

        def scat_wait(idst_v, r0):
            for r in range(r0, r0 + GR):
                pltpu.make_async_copy(rows_v.at[pl.ds(r * 128, 128)],
                                      acc.at[idst_v.at[r]], sems).wait()

        def one_chunk(k, isrc_v, idst_v):
            load_idx(k, isrc_v, idst_v)
            gath(isrc_v, 0)
            gath_wait(isrc_v, 0)
            gath(isrc_v, GR)
            scat(idst_v, 0)
            gath_wait(isrc_v, GR)
            scat_wait(idst_v, 0)
            scat(idst_v, GR)
            scat_wait(idst_v, GR)

        def body(m, deferred):
            kA = s + 32 * m
            load_idx_async(kA, isrc_a, idst_a)
            if deferred:
                scat_wait(idst_b, GR)
            load_idx_async(kA + 16, isrc_b, idst_b)
            load_idx_wait(kA, isrc_a, idst_a)
            load_idx_wait(kA + 16, isrc_b, idst_b)
            gath(isrc_a, 0)
            gath_wait(isrc_a, 0)
            scat(idst_a, 0)
            gath(isrc_a, GR)
            gath_wait(isrc_a, GR)
            scat_wait(idst_a, 0)
            scat(idst_a, GR)
            gath(isrc_b, 0)
            gath_wait(isrc_b, 0)
            scat_wait(idst_a, GR)
            scat(idst_b, 0)
            gath(isrc_b, GR)
            gath_wait(isrc_b, GR)
            scat_wait(idst_b, 0)
            scat(idst_b, GR)
            return 0

        body(0, False)
        lax.fori_loop(1, 30, lambda m, _: body(m, True), 0)
        scat_wait(idst_b, GR)
        one_chunk(s + 960, isrc_a, idst_a)

        @pl.when(s == 0)
        def _():
            one_chunk(976, isrc_a, idst_a)
        plsc.subcore_barrier()
        off = 0
        while off < per_tile:
            nd = min(CHUNK, per_tile - off)
            row0 = s * per_tile + off
            pltpu.sync_copy(acc.at[pl.ds(row0, nd)], rows_v.at[pl.ds(0, nd)])
            pltpu.sync_copy(rows_v.at[pl.ds(0, nd)],
                            out_hbm.at[pl.ds(row0, nd)])
            pltpu.sync_copy(inv_hbm.at[pl.ds(row0, nd)], inv_v.at[pl.ds(0, nd)])

            def g_body(g, _):
                fvec = inv_v[pl.ds(g * 16, 16)]
                f2 = fvec * fvec
                rowidx = lax.iota(jnp.int32, 16) + g * 16
                for col in range(HH):
                    colidx = jnp.full((16,), col, jnp.int32)
                    v = plsc.load_gather(rows_v, [rowidx, colidx]) * f2
                    plsc.store_scatter(rows_v, [rowidx, colidx], v)
                return 0

            lax.fori_loop(0, nd // 16, g_body, 0)
            pltpu.sync_copy(rows_v.at[pl.ds(0, nd)],
                            scaled_hbm.at[pl.ds(row0, nd)])
            off += nd
        plsc.subcore_barrier()

    @pl.when(c == 0)
    def _():
        direction(ss_lo, rs_hbm, ru_hbm, rawu_lo, inv_u_hbm, sun_lo,
                  N_PER_TILE)
        direction(su_lo, ru_hbm, rs_hbm, raws_lo, inv_s_hbm, ssn_lo,
                  M_PER_TILE)

    @pl.when(c == 1)
    def _():
        direction(ss_hi, rs_hbm, ru_hbm, rawu_hi, inv_u_hbm, sun_hi,
                  N_PER_TILE)
        direction(su_hi, ru_hbm, rs_hbm, raws_hi, inv_s_hbm, ssn_hi,
                  M_PER_TILE)


_layer_call = functools.partial(
    pl.kernel,
    out_type=(jax.ShapeDtypeStruct((N_PAD, HH), jnp.float32),
              jax.ShapeDtypeStruct((N_PAD, HH), jnp.float32),
              jax.ShapeDtypeStruct((M_PAD, HH), jnp.float32),
              jax.ShapeDtypeStruct((M_PAD, HH), jnp.float32),
              jax.ShapeDtypeStruct((N_PAD, HH), jnp.float32),
              jax.ShapeDtypeStruct((N_PAD, HH), jnp.float32),
              jax.ShapeDtypeStruct((M_PAD, HH), jnp.float32),
              jax.ShapeDtypeStruct((M_PAD, HH), jnp.float32)),
    mesh=_mesh,
    scratch_types=[
        pltpu.VMEM((ROWS_PER_CHUNK, 128), jnp.int32),
        pltpu.VMEM((ROWS_PER_CHUNK, 128), jnp.int32),
        pltpu.VMEM((ROWS_PER_CHUNK, 128), jnp.int32),
        pltpu.VMEM((ROWS_PER_CHUNK, 128), jnp.int32),
        pltpu.VMEM((CHUNK, HH), jnp.float32),
        pltpu.VMEM((CHUNK,), jnp.float32),
        pltpu.VMEM_SHARED((M_PAD, HH), jnp.float32),
        pltpu.SemaphoreType.DMA,
        pltpu.SemaphoreType.DMA,
        pltpu.SemaphoreType.DMA,
    ],
    compiler_params=_sc_params_nl,
)(_layer_body)


def kernel(spot_weight, user_weight, user_spot):
    row_u = user_spot[0]
    row_s = user_spot[1]
    pad = E_PAD - N_EDGES
    ru2d = jnp.concatenate(
        [row_u, jnp.full((pad,), N_USER, jnp.int32)]).reshape(NIDXROWS, 128)
    rs2d = jnp.concatenate(
        [row_s, jnp.full((pad,), M_SPOT, jnp.int32)]).reshape(NIDXROWS, 128)

    uw_pad = jnp.pad(user_weight, ((0, N_PAD - N_USER), (0, 0)))
    sw_pad = jnp.pad(spot_weight, ((0, M_PAD - M_SPOT), (0, 0)))
    zeros2d = jnp.zeros((CHUNK, HH), jnp.float32)

    du, ds = _counts_call(ru2d, rs2d)
    inv_u = jnp.where(du > 0, lax.rsqrt(du), 0.0)
    inv_s = jnp.where(ds > 0, lax.rsqrt(ds), 0.0)

    su = uw_pad * inv_u[:, None]
    ss = sw_pad * inv_s[:, None]
    su_lo, su_hi = su[:, :HH], su[:, HH:]
    ss_lo, ss_hi = ss[:, :HH], ss[:, HH:]

    raws_u, raws_s = [], []
    for _ in range(NUM_LAYERS):
        (rawu_lo, rawu_hi, raws_lo, raws_hi,
         su_lo, su_hi, ss_lo, ss_hi) = _layer_call(
            ru2d, rs2d, ss_lo, ss_hi, su_lo, su_hi, inv_u, inv_s, zeros2d)
        raws_u.append((rawu_lo, rawu_hi))
        raws_s.append((raws_lo, raws_hi))

    outs_u = [(jnp.concatenate(r, axis=1) * inv_u[:, None])[:N_USER]
              for r in raws_u]
    outs_s = [(jnp.concatenate(r, axis=1) * inv_s[:, None])[:M_SPOT]
              for r in raws_s]

    user_stack = jnp.stack([user_weight] + outs_u, axis=1)
    spot_stack = jnp.stack([spot_weight] + outs_s, axis=1)
    return (spot_stack, user_stack)


# speedup vs baseline: 1.1779x; 1.1779x over previous
import functools

import jax
import jax.numpy as jnp
from jax import lax
from jax.experimental import pallas as pl
from jax.experimental.pallas import tpu as pltpu
from jax.experimental.pallas import tpu_sc as plsc

N_USER = 27094
M_SPOT = 42852
HIDDEN = 64
HH = 32
NUM_LAYERS = 3
N_EDGES = 1000000

N_PAD = 27136
M_PAD = 43008
N_PER_TILE = N_PAD // 16
M_PER_TILE = M_PAD // 16

ROWS_PER_CHUNK = 8
CHUNK = ROWS_PER_CHUNK * 128
NCHUNK = -(-N_EDGES // CHUNK)
E_PAD = NCHUNK * CHUNK
NIDXROWS = NCHUNK * ROWS_PER_CHUNK
ITERS_PER_TILE = -(-NCHUNK // 16)
ZROWS = CHUNK

_mesh = plsc.VectorSubcoreMesh(core_axis_name="c", subcore_axis_name="s")
_sc_params = pltpu.CompilerParams(use_tc_tiling_on_sc=False)
_sc_params_nl = pltpu.CompilerParams(use_tc_tiling_on_sc=False,
                                     needs_layout_passes=False)


def _counts_body(ru_hbm, rs_hbm, du_out, ds_out, idx_v, ones_v, zb_v, acc, sem):
    c = lax.axis_index("c")
    s = lax.axis_index("s")

    def fill(ref, n, val):
        def b(i, _):
            ref[pl.ds(i * 16, 16)] = jnp.full((16,), val, jnp.float32)
            return 0
        lax.fori_loop(0, n // 16, b, 0)

    fill(ones_v, 128, 1.0)
    fill(zb_v, M_PER_TILE, 0.0)

    def count_into(idx_hbm, out_hbm, per_tile):
        pltpu.sync_copy(zb_v.at[pl.ds(0, per_tile)],
                        acc.at[pl.ds(s * per_tile, per_tile)])
        plsc.subcore_barrier()

        def chunk_body(j, _):
            k = s + 16 * j

            @pl.when(k < NCHUNK)
            def _():
                pltpu.sync_copy(
                    idx_hbm.at[pl.ds(k * ROWS_PER_CHUNK, ROWS_PER_CHUNK)],
                    idx_v)
                for r in range(ROWS_PER_CHUNK):
                    pltpu.sync_copy(ones_v, acc.at[idx_v.at[r]], add=True)
            return 0

        lax.fori_loop(0, ITERS_PER_TILE, chunk_body, 0)
        plsc.subcore_barrier()
        off = 0
        while off < per_tile:
            nd = min(M_PER_TILE, per_tile - off)
            pltpu.sync_copy(acc.at[pl.ds(s * per_tile + off, nd)],
                            zb_v.at[pl.ds(0, nd)])
            pltpu.sync_copy(zb_v.at[pl.ds(0, nd)],
                            out_hbm.at[pl.ds(s * per_tile + off, nd)])
            off += nd

    @pl.when(c == 0)
    def _():
        count_into(ru_hbm, du_out, N_PER_TILE)

    @pl.when(c == 1)
    def _():
        count_into(rs_hbm, ds_out, M_PER_TILE)


_counts_call = functools.partial(
    pl.kernel,
    out_type=(jax.ShapeDtypeStruct((N_PAD,), jnp.float32),
              jax.ShapeDtypeStruct((M_PAD,), jnp.float32)),
    mesh=_mesh,
    scratch_types=[
        pltpu.VMEM((ROWS_PER_CHUNK, 128), jnp.int32),
        pltpu.VMEM((128,), jnp.float32),
        pltpu.VMEM((M_PER_TILE,), jnp.float32),
        pltpu.VMEM_SHARED((M_PAD,), jnp.float32),
        pltpu.SemaphoreType.DMA,
    ],
    compiler_params=_sc_params,
)(_counts_body)


def _layer_body(ru_hbm, rs_hbm, ss_lo, ss_hi, su_lo, su_hi,
                inv_u_hbm, inv_s_hbm, zeros2d,
                rawu_lo, rawu_hi, raws_lo, raws_hi,
                sun_lo, sun_hi, ssn_lo, ssn_hi,
                isrc_a, idst_a, isrc_b, idst_b, rows_v, inv_v,
                acc, semg, sems, semi):
    c = lax.axis_index("c")
    s = lax.axis_index("s")
    GR = ROWS_PER_CHUNK // 2

    def direction(src_tbl, isrc_hbm, idst_hbm, out_hbm, inv_hbm, scaled_hbm,
                  per_tile):
        pltpu.sync_copy(zeros2d, rows_v)
        off = 0
        while off < per_tile:
            nz = min(CHUNK, per_tile - off)
            pltpu.sync_copy(rows_v.at[pl.ds(0, nz)],
                            acc.at[pl.ds(s * per_tile + off, nz)])
            off += nz
        plsc.subcore_barrier()

        def load_idx(k, isrc_v, idst_v):
            pltpu.sync_copy(
                isrc_hbm.at[pl.ds(k * ROWS_PER_CHUNK, ROWS_PER_CHUNK)],
                isrc_v)
            pltpu.sync_copy(
                idst_hbm.at[pl.ds(k * ROWS_PER_CHUNK, ROWS_PER_CHUNK)],
                idst_v)

        def load_idx_async(k, isrc_v, idst_v):
            pltpu.async_copy(
                isrc_hbm.at[pl.ds(k * ROWS_PER_CHUNK, ROWS_PER_CHUNK)],
                isrc_v, semi)
            pltpu.async_copy(
                idst_hbm.at[pl.ds(k * ROWS_PER_CHUNK, ROWS_PER_CHUNK)],
                idst_v, semi)

        def load_idx_wait(k, isrc_v, idst_v):
            pltpu.make_async_copy(
                isrc_hbm.at[pl.ds(k * ROWS_PER_CHUNK, ROWS_PER_CHUNK)],
                isrc_v, semi).wait()
            pltpu.make_async_copy(
                idst_hbm.at[pl.ds(k * ROWS_PER_CHUNK, ROWS_PER_CHUNK)],
                idst_v, semi).wait()

        def gath(isrc_v, r0):
            for r in range(r0, r0 + GR):
                pltpu.async_copy(src_tbl.at[isrc_v.at[r]],
                                 rows_v.at[pl.ds(r * 128, 128)], semg)

        def gath_wait(isrc_v, r0):
            for r in range(r0, r0 + GR):
                pltpu.make_async_copy(src_tbl.at[isrc_v.at[r]],
                                      rows_v.at[pl.ds(r * 128, 128)],
                                      semg).wait()

        def scat(idst_v, r0):
            for r in range(r0, r0 + GR):
                pltpu.async_copy(rows_v.at[pl.ds(r * 128, 128)],
                                 acc.at[idst_v.at[r]], sems, add=True)

        def scat_wait(idst_v, r0):
            for r in range(r0, r0 + GR):
                pltpu.make_async_copy(rows_v.at[pl.ds(r * 128, 128)],
                                      acc.at[idst_v.at[r]], sems).wait()

        def one_chunk(k, isrc_v, idst_v):
            load_idx(k, isrc_v, idst_v)
            gath(isrc_v, 0)
            gath_wait(isrc_v, 0)
            gath(isrc_v, GR)
            scat(idst_v, 0)
            gath_wait(isrc_v, GR)
            scat_wait(idst_v, 0)
            scat(idst_v, GR)
            scat_wait(idst_v, GR)

        def body(m, deferred):
            kA = s + 32 * m
            load_idx_async(kA, isrc_a, idst_a)
            if deferred:
                scat_wait(idst_b, GR)
            load_idx_async(kA + 16, isrc_b, idst_b)
            load_idx_wait(kA, isrc_a, idst_a)
            load_idx_wait(kA + 16, isrc_b, idst_b)
            gath(isrc_a, 0)
            gath_wait(isrc_a, 0)
            scat(idst_a, 0)
            gath(isrc_a, GR)
            gath_wait(isrc_a, GR)
            scat_wait(idst_a, 0)
            scat(idst_a, GR)
            gath(isrc_b, 0)
            gath_wait(isrc_b, 0)
            scat_wait(idst_a, GR)
            scat(idst_b, 0)
            gath(isrc_b, GR)
            gath_wait(isrc_b, GR)
            scat_wait(idst_b, 0)
            scat(idst_b, GR)
            return 0

        body(0, False)
        lax.fori_loop(1, 30, lambda m, _: body(m, True), 0)
        scat_wait(idst_b, GR)
        one_chunk(s + 960, isrc_a, idst_a)

        @pl.when(s == 0)
        def _():
            one_chunk(976, isrc_a, idst_a)
        plsc.subcore_barrier()
        off = 0
        while off < per_tile:
            nd = min(CHUNK, per_tile - off)
            row0 = s * per_tile + off
            pltpu.sync_copy(acc.at[pl.ds(row0, nd)], rows_v.at[pl.ds(0, nd)])
            pltpu.sync_copy(rows_v.at[pl.ds(0, nd)],
                            out_hbm.at[pl.ds(row0, nd)])
            pltpu.sync_copy(inv_hbm.at[pl.ds(row0, nd)], inv_v.at[pl.ds(0, nd)])

            def g_body(g, _):
                fvec = inv_v[pl.ds(g * 16, 16)]
                f2 = fvec * fvec
                for j in range(16):
                    i = g * 16 + j
                    f = f2[j]
                    rowidx = jnp.full((16,), i, jnp.int32)
                    for h in (0, 16):
                        colidx = lax.iota(jnp.int32, 16) + h
                        v = plsc.load_gather(rows_v, [rowidx, colidx]) * f
                        plsc.store_scatter(rows_v, [rowidx, colidx], v)
                return 0

            lax.fori_loop(0, nd // 16, g_body, 0)
            pltpu.sync_copy(rows_v.at[pl.ds(0, nd)],
                            scaled_hbm.at[pl.ds(row0, nd)])
            off += nd
        plsc.subcore_barrier()

    @pl.when(c == 0)
    def _():
        direction(ss_lo, rs_hbm, ru_hbm, rawu_lo, inv_u_hbm, sun_lo,
                  N_PER_TILE)
        direction(su_lo, ru_hbm, rs_hbm, raws_lo, inv_s_hbm, ssn_lo,
                  M_PER_TILE)

    @pl.when(c == 1)
    def _():
        direction(ss_hi, rs_hbm, ru_hbm, rawu_hi, inv_u_hbm, sun_hi,
                  N_PER_TILE)
        direction(su_hi, ru_hbm, rs_hbm, raws_hi, inv_s_hbm, ssn_hi,
                  M_PER_TILE)


_layer_call = functools.partial(
    pl.kernel,
    out_type=(jax.ShapeDtypeStruct((N_PAD, HH), jnp.float32),
              jax.ShapeDtypeStruct((N_PAD, HH), jnp.float32),
              jax.ShapeDtypeStruct((M_PAD, HH), jnp.float32),
              jax.ShapeDtypeStruct((M_PAD, HH), jnp.float32),
              jax.ShapeDtypeStruct((N_PAD, HH), jnp.float32),
              jax.ShapeDtypeStruct((N_PAD, HH), jnp.float32),
              jax.ShapeDtypeStruct((M_PAD, HH), jnp.float32),
              jax.ShapeDtypeStruct((M_PAD, HH), jnp.float32)),
    mesh=_mesh,
    scratch_types=[
        pltpu.VMEM((ROWS_PER_CHUNK, 128), jnp.int32),
        pltpu.VMEM((ROWS_PER_CHUNK, 128), jnp.int32),
        pltpu.VMEM((ROWS_PER_CHUNK, 128), jnp.int32),
        pltpu.VMEM((ROWS_PER_CHUNK, 128), jnp.int32),
        pltpu.VMEM((CHUNK, HH), jnp.float32),
        pltpu.VMEM((CHUNK,), jnp.float32),
        pltpu.VMEM_SHARED((M_PAD, HH), jnp.float32),
        pltpu.SemaphoreType.DMA,
        pltpu.SemaphoreType.DMA,
        pltpu.SemaphoreType.DMA,
    ],
    compiler_params=_sc_params_nl,
)(_layer_body)


def kernel(spot_weight, user_weight, user_spot):
    row_u = user_spot[0]
    row_s = user_spot[1]
    pad = E_PAD - N_EDGES
    ru2d = jnp.concatenate(
        [row_u, jnp.full((pad,), N_USER, jnp.int32)]).reshape(NIDXROWS, 128)
    rs2d = jnp.concatenate(
        [row_s, jnp.full((pad,), M_SPOT, jnp.int32)]).reshape(NIDXROWS, 128)

    uw_pad = jnp.pad(user_weight, ((0, N_PAD - N_USER), (0, 0)))
    sw_pad = jnp.pad(spot_weight, ((0, M_PAD - M_SPOT), (0, 0)))
    zeros2d = jnp.zeros((CHUNK, HH), jnp.float32)

    du, ds = _counts_call(ru2d, rs2d)
    inv_u = jnp.where(du > 0, lax.rsqrt(du), 0.0)
    inv_s = jnp.where(ds > 0, lax.rsqrt(ds), 0.0)

    su = uw_pad * inv_u[:, None]
    ss = sw_pad * inv_s[:, None]
    su_lo, su_hi = su[:, :HH], su[:, HH:]
    ss_lo, ss_hi = ss[:, :HH], ss[:, HH:]

    raws_u, raws_s = [], []
    for _ in range(NUM_LAYERS):
        (rawu_lo, rawu_hi, raws_lo, raws_hi,
         su_lo, su_hi, ss_lo, ss_hi) = _layer_call(
            ru2d, rs2d, ss_lo, ss_hi, su_lo, su_hi, inv_u, inv_s, zeros2d)
        raws_u.append((rawu_lo, rawu_hi))
        raws_s.append((raws_lo, raws_hi))

    outs_u = [(jnp.concatenate(r, axis=1) * inv_u[:, None])[:N_USER]
              for r in raws_u]
    outs_s = [(jnp.concatenate(r, axis=1) * inv_s[:, None])[:M_SPOT]
              for r in raws_s]

    user_stack = jnp.stack([user_weight] + outs_u, axis=1)
    spot_stack = jnp.stack([spot_weight] + outs_s, axis=1)
    return (spot_stack, user_stack)


# speedup vs baseline: 1.2004x; 1.0191x over previous
import functools

import jax
import jax.numpy as jnp
from jax import lax
from jax.experimental import pallas as pl
from jax.experimental.pallas import tpu as pltpu
from jax.experimental.pallas import tpu_sc as plsc

N_USER = 27094
M_SPOT = 42852
HIDDEN = 64
HH = 32
NUM_LAYERS = 3
N_EDGES = 1000000

N_PAD = 27136
M_PAD = 43008
N_PER_TILE = N_PAD // 16
M_PER_TILE = M_PAD // 16

ROWS_PER_CHUNK = 8
CHUNK = ROWS_PER_CHUNK * 128
NCHUNK = -(-N_EDGES // CHUNK)
E_PAD = NCHUNK * CHUNK
NIDXROWS = NCHUNK * ROWS_PER_CHUNK
ITERS_PER_TILE = -(-NCHUNK // 16)
ZROWS = CHUNK

_mesh = plsc.VectorSubcoreMesh(core_axis_name="c", subcore_axis_name="s")
_sc_params = pltpu.CompilerParams(use_tc_tiling_on_sc=False)
_sc_params_nl = pltpu.CompilerParams(use_tc_tiling_on_sc=False,
                                     needs_layout_passes=False)


def _counts_body(ru_hbm, rs_hbm, du_out, ds_out, idx_a, idx_b, ones_v, zb_v,
                 acc, semi, sema, semb):
    c = lax.axis_index("c")
    s = lax.axis_index("s")

    def fill(ref, n, val):
        def b(i, _):
            ref[pl.ds(i * 16, 16)] = jnp.full((16,), val, jnp.float32)
            return 0
        lax.fori_loop(0, n // 16, b, 0)

    fill(ones_v, 128, 1.0)
    fill(zb_v, M_PER_TILE, 0.0)

    def count_into(idx_hbm, out_hbm, per_tile):
        pltpu.sync_copy(zb_v.at[pl.ds(0, per_tile)],
                        acc.at[pl.ds(s * per_tile, per_tile)])
        plsc.subcore_barrier()

        def load_async(k, idx_v):
            pltpu.async_copy(
                idx_hbm.at[pl.ds(k * ROWS_PER_CHUNK, ROWS_PER_CHUNK)],
                idx_v, semi)

        def load_wait(k, idx_v):
            pltpu.make_async_copy(
                idx_hbm.at[pl.ds(k * ROWS_PER_CHUNK, ROWS_PER_CHUNK)],
                idx_v, semi).wait()

        def scats(idx_v, sem):
            for r in range(ROWS_PER_CHUNK):
                pltpu.async_copy(ones_v, acc.at[idx_v.at[r]], sem, add=True)

        def scats_wait(idx_v, sem):
            for r in range(ROWS_PER_CHUNK):
                pltpu.make_async_copy(ones_v, acc.at[idx_v.at[r]], sem).wait()

        def one_chunk(k, idx_v):
            pltpu.sync_copy(
                idx_hbm.at[pl.ds(k * ROWS_PER_CHUNK, ROWS_PER_CHUNK)], idx_v)
            scats(idx_v, sema)
            scats_wait(idx_v, sema)

        def body(m, deferred):
            kA = s + 32 * m
            load_async(kA, idx_a)
            if deferred:
                scats_wait(idx_b, semb)
            load_async(kA + 16, idx_b)
            load_wait(kA, idx_a)
            load_wait(kA + 16, idx_b)
            scats(idx_a, sema)
            scats(idx_b, semb)
            scats_wait(idx_a, sema)
            return 0

        body(0, False)
        lax.fori_loop(1, 30, lambda m, _: body(m, True), 0)
        scats_wait(idx_b, semb)
        one_chunk(s + 960, idx_a)

        @pl.when(s == 0)
        def _():
            one_chunk(976, idx_a)
        plsc.subcore_barrier()
        off = 0
        while off < per_tile:
            nd = min(M_PER_TILE, per_tile - off)
            pltpu.sync_copy(acc.at[pl.ds(s * per_tile + off, nd)],
                            zb_v.at[pl.ds(0, nd)])
            pltpu.sync_copy(zb_v.at[pl.ds(0, nd)],
                            out_hbm.at[pl.ds(s * per_tile + off, nd)])
            off += nd

    @pl.when(c == 0)
    def _():
        count_into(ru_hbm, du_out, N_PER_TILE)

    @pl.when(c == 1)
    def _():
        count_into(rs_hbm, ds_out, M_PER_TILE)


_counts_call = functools.partial(
    pl.kernel,
    out_type=(jax.ShapeDtypeStruct((N_PAD,), jnp.float32),
              jax.ShapeDtypeStruct((M_PAD,), jnp.float32)),
    mesh=_mesh,
    scratch_types=[
        pltpu.VMEM((ROWS_PER_CHUNK, 128), jnp.int32),
        pltpu.VMEM((ROWS_PER_CHUNK, 128), jnp.int32),
        pltpu.VMEM((128,), jnp.float32),
        pltpu.VMEM((M_PER_TILE,), jnp.float32),
        pltpu.VMEM_SHARED((M_PAD,), jnp.float32),
        pltpu.SemaphoreType.DMA,
        pltpu.SemaphoreType.DMA,
        pltpu.SemaphoreType.DMA,
    ],
    compiler_params=_sc_params,
)(_counts_body)


def _layer_body(ru_hbm, rs_hbm, ss_lo, ss_hi, su_lo, su_hi,
                inv_u_hbm, inv_s_hbm, zeros2d,
                rawu_lo, rawu_hi, raws_lo, raws_hi,
                sun_lo, sun_hi, ssn_lo, ssn_hi,
                isrc_a, idst_a, isrc_b, idst_b, rows_v, inv_v,
                acc, semg, sems, semi):
    c = lax.axis_index("c")
    s = lax.axis_index("s")
    GR = ROWS_PER_CHUNK // 2

    def direction(src_tbl, isrc_hbm, idst_hbm, out_hbm, inv_hbm, scaled_hbm,
                  per_tile):
        pltpu.sync_copy(zeros2d, rows_v)
        off = 0
        while off < per_tile:
            nz = min(CHUNK, per_tile - off)
            pltpu.sync_copy(rows_v.at[pl.ds(0, nz)],
                            acc.at[pl.ds(s * per_tile + off, nz)])
            off += nz
        plsc.subcore_barrier()

        def load_idx(k, isrc_v, idst_v):
            pltpu.sync_copy(
                isrc_hbm.at[pl.ds(k * ROWS_PER_CHUNK, ROWS_PER_CHUNK)],
                isrc_v)
            pltpu.sync_copy(
                idst_hbm.at[pl.ds(k * ROWS_PER_CHUNK, ROWS_PER_CHUNK)],
                idst_v)

        def load_idx_async(k, isrc_v, idst_v):
            pltpu.async_copy(
                isrc_hbm.at[pl.ds(k * ROWS_PER_CHUNK, ROWS_PER_CHUNK)],
                isrc_v, semi)
            pltpu.async_copy(
                idst_hbm.at[pl.ds(k * ROWS_PER_CHUNK, ROWS_PER_CHUNK)],
                idst_v, semi)

        def load_idx_wait(k, isrc_v, idst_v):
            pltpu.make_async_copy(
                isrc_hbm.at[pl.ds(k * ROWS_PER_CHUNK, ROWS_PER_CHUNK)],
                isrc_v, semi).wait()
            pltpu.make_async_copy(
                idst_hbm.at[pl.ds(k * ROWS_PER_CHUNK, ROWS_PER_CHUNK)],
                idst_v, semi).wait()

        def gath(isrc_v, r0):
            for r in range(r0, r0 + GR):
                pltpu.async_copy(src_tbl.at[isrc_v.at[r]],
                                 rows_v.at[pl.ds(r * 128, 128)], semg)

        def gath_wait(isrc_v, r0):
            for r in range(r0, r0 + GR):
                pltpu.make_async_copy(src_tbl.at[isrc_v.at[r]],
                                      rows_v.at[pl.ds(r * 128, 128)],
                                      semg).wait()

        def scat(idst_v, r0):
            for r in range(r0, r0 + GR):
                pltpu.async_copy(rows_v.at[pl.ds(r * 128, 128)],
                                 acc.at[idst_v.at[r]], sems, add=True)

        def scat_wait(idst_v, r0):
            for r in range(r0, r0 + GR):
                pltpu.make_async_copy(rows_v.at[pl.ds(r * 128, 128)],
                                      acc.at[idst_v.at[r]], sems).wait()

        def one_chunk(k, isrc_v, idst_v):
            load_idx(k, isrc_v, idst_v)
            gath(isrc_v, 0)
            gath_wait(isrc_v, 0)
            gath(isrc_v, GR)
            scat(idst_v, 0)
            gath_wait(isrc_v, GR)
            scat_wait(idst_v, 0)
            scat(idst_v, GR)
            scat_wait(idst_v, GR)

        def body(m, deferred):
            kA = s + 32 * m
            load_idx_async(kA, isrc_a, idst_a)
            if deferred:
                scat_wait(idst_b, GR)
            load_idx_async(kA + 16, isrc_b, idst_b)
            load_idx_wait(kA, isrc_a, idst_a)
            load_idx_wait(kA + 16, isrc_b, idst_b)
            gath(isrc_a, 0)
            gath_wait(isrc_a, 0)
            scat(idst_a, 0)
            gath(isrc_a, GR)
            gath_wait(isrc_a, GR)
            scat_wait(idst_a, 0)
            scat(idst_a, GR)
            gath(isrc_b, 0)
            gath_wait(isrc_b, 0)
            scat_wait(idst_a, GR)
            scat(idst_b, 0)
            gath(isrc_b, GR)
            gath_wait(isrc_b, GR)
            scat_wait(idst_b, 0)
            scat(idst_b, GR)
            return 0

        body(0, False)
        lax.fori_loop(1, 30, lambda m, _: body(m, True), 0)
        scat_wait(idst_b, GR)
        one_chunk(s + 960, isrc_a, idst_a)

        @pl.when(s == 0)
        def _():
            one_chunk(976, isrc_a, idst_a)
        plsc.subcore_barrier()
        off = 0
        while off < per_tile:
            nd = min(CHUNK, per_tile - off)
            row0 = s * per_tile + off
            pltpu.sync_copy(acc.at[pl.ds(row0, nd)], rows_v.at[pl.ds(0, nd)])
            pltpu.sync_copy(rows_v.at[pl.ds(0, nd)],
                            out_hbm.at[pl.ds(row0, nd)])
            pltpu.sync_copy(inv_hbm.at[pl.ds(row0, nd)], inv_v.at[pl.ds(0, nd)])

            def g_body(g, _):
                fvec = inv_v[pl.ds(g * 16, 16)]
                f2 = fvec * fvec
                for j in range(16):
                    i = g * 16 + j
                    f = f2[j]
                    rowidx = jnp.full((16,), i, jnp.int32)
                    for h in (0, 16):
                        colidx = lax.iota(jnp.int32, 16) + h
                        v = plsc.load_gather(rows_v, [rowidx, colidx]) * f
                        plsc.store_scatter(rows_v, [rowidx, colidx], v)
                return 0

            lax.fori_loop(0, nd // 16, g_body, 0)
            pltpu.sync_copy(rows_v.at[pl.ds(0, nd)],
                            scaled_hbm.at[pl.ds(row0, nd)])
            off += nd
        plsc.subcore_barrier()

    @pl.when(c == 0)
    def _():
        direction(ss_lo, rs_hbm, ru_hbm, rawu_lo, inv_u_hbm, sun_lo,
                  N_PER_TILE)
        direction(su_lo, ru_hbm, rs_hbm, raws_lo, inv_s_hbm, ssn_lo,
                  M_PER_TILE)

    @pl.when(c == 1)
    def _():
        direction(ss_hi, rs_hbm, ru_hbm, rawu_hi, inv_u_hbm, sun_hi,
                  N_PER_TILE)
        direction(su_hi, ru_hbm, rs_hbm, raws_hi, inv_s_hbm, ssn_hi,
                  M_PER_TILE)


_layer_call = functools.partial(
    pl.kernel,
    out_type=(jax.ShapeDtypeStruct((N_PAD, HH), jnp.float32),
              jax.ShapeDtypeStruct((N_PAD, HH), jnp.float32),
              jax.ShapeDtypeStruct((M_PAD, HH), jnp.float32),
              jax.ShapeDtypeStruct((M_PAD, HH), jnp.float32),
              jax.ShapeDtypeStruct((N_PAD, HH), jnp.float32),
              jax.ShapeDtypeStruct((N_PAD, HH), jnp.float32),
              jax.ShapeDtypeStruct((M_PAD, HH), jnp.float32),
              jax.ShapeDtypeStruct((M_PAD, HH), jnp.float32)),
    mesh=_mesh,
    scratch_types=[
        pltpu.VMEM((ROWS_PER_CHUNK, 128), jnp.int32),
        pltpu.VMEM((ROWS_PER_CHUNK, 128), jnp.int32),
        pltpu.VMEM((ROWS_PER_CHUNK, 128), jnp.int32),
        pltpu.VMEM((ROWS_PER_CHUNK, 128), jnp.int32),
        pltpu.VMEM((CHUNK, HH), jnp.float32),
        pltpu.VMEM((CHUNK,), jnp.float32),
        pltpu.VMEM_SHARED((M_PAD, HH), jnp.float32),
        pltpu.SemaphoreType.DMA,
        pltpu.SemaphoreType.DMA,
        pltpu.SemaphoreType.DMA,
    ],
    compiler_params=_sc_params_nl,
)(_layer_body)


def kernel(spot_weight, user_weight, user_spot):
    row_u = user_spot[0]
    row_s = user_spot[1]
    pad = E_PAD - N_EDGES
    ru2d = jnp.concatenate(
        [row_u, jnp.full((pad,), N_USER, jnp.int32)]).reshape(NIDXROWS, 128)
    rs2d = jnp.concatenate(
        [row_s, jnp.full((pad,), M_SPOT, jnp.int32)]).reshape(NIDXROWS, 128)

    uw_pad = jnp.pad(user_weight, ((0, N_PAD - N_USER), (0, 0)))
    sw_pad = jnp.pad(spot_weight, ((0, M_PAD - M_SPOT), (0, 0)))
    zeros2d = jnp.zeros((CHUNK, HH), jnp.float32)

    du, ds = _counts_call(ru2d, rs2d)
    inv_u = jnp.where(du > 0, lax.rsqrt(du), 0.0)
    inv_s = jnp.where(ds > 0, lax.rsqrt(ds), 0.0)

    su = uw_pad * inv_u[:, None]
    ss = sw_pad * inv_s[:, None]
    su_lo, su_hi = su[:, :HH], su[:, HH:]
    ss_lo, ss_hi = ss[:, :HH], ss[:, HH:]

    raws_u, raws_s = [], []
    for _ in range(NUM_LAYERS):
        (rawu_lo, rawu_hi, raws_lo, raws_hi,
         su_lo, su_hi, ss_lo, ss_hi) = _layer_call(
            ru2d, rs2d, ss_lo, ss_hi, su_lo, su_hi, inv_u, inv_s, zeros2d)
        raws_u.append((rawu_lo, rawu_hi))
        raws_s.append((raws_lo, raws_hi))

    outs_u = [(jnp.concatenate(r, axis=1) * inv_u[:, None])[:N_USER]
              for r in raws_u]
    outs_s = [(jnp.concatenate(r, axis=1) * inv_s[:, None])[:M_SPOT]
              for r in raws_s]

    user_stack = jnp.stack([user_weight] + outs_u, axis=1)
    spot_stack = jnp.stack([spot_weight] + outs_s, axis=1)
    return (spot_stack, user_stack)
